# baseline (speedup 1.0000x reference)
import functools

import jax
import jax.numpy as jnp
from jax import lax
from jax.experimental import pallas as pl
from jax.experimental.pallas import tpu as pltpu
from jax.experimental.pallas import tpu_sc as plsc

B = 16384
ACTION_DIM = 256
H = 1024
D = 256
K = 8192
BETA = 0.25

CB_CHUNKS = 8
CB_ROWS = K // CB_CHUNKS

NC = 2
NS = 16
L = 16
NW = NC * NS
ROWS_PER_W = B // NW
SC_CHUNK = 64
SC_NCHUNK = ROWS_PER_W // SC_CHUNK


def _sc_reduce_body(a_hbm, out_hbm, buf, acc, sems):
    wid = lax.axis_index("s") * NC + lax.axis_index("c")
    base = wid * ROWS_PER_W

    zero = jnp.zeros((L,), jnp.float32)
    for j in range(ACTION_DIM // L):
        acc[0, pl.ds(j * L, L)] = zero
        acc[1, pl.ds(j * L, L)] = zero

    def chunk_copy(c, b):
        return pltpu.make_async_copy(
            a_hbm.at[pl.ds(base + c * SC_CHUNK, SC_CHUNK), :],
            buf.at[b], sems.at[b])

    chunk_copy(0, 0).start()
    chunk_copy(1, 1).start()
    for c in range(SC_NCHUNK):
        b = c % 2
        chunk_copy(c, b).wait()
        for j in range(ACTION_DIM // L):
            sl = pl.ds(j * L, L)

            def row_body(r, carry, b=b, sl=sl):
                cs, sq = carry
                v = buf[b, r, sl]
                return cs + v, sq + v * v

            cs, sq = lax.fori_loop(0, SC_CHUNK, row_body,
                                   (acc[0, sl], acc[1, sl]), unroll=8)
            acc[0, sl] = cs
            acc[1, sl] = sq
        if c + 2 < SC_NCHUNK:
            chunk_copy(c + 2, b).start()

    pltpu.sync_copy(acc, out_hbm.at[wid])


def _sc_reduce(action):
    return pl.kernel(
        _sc_reduce_body,
        mesh=plsc.VectorSubcoreMesh(core_axis_name="c", subcore_axis_name="s"),
        out_type=jax.ShapeDtypeStruct((NW, 2, ACTION_DIM), jnp.float32),
        scratch_types=[
            pltpu.VMEM((2, SC_CHUNK, ACTION_DIM), jnp.float32),
            pltpu.VMEM((2, ACTION_DIM), jnp.float32),
            pltpu.SemaphoreType.DMA((2,)),
        ],
    )(action)


def _tc_main_body(x_ref, b1_ref, b2_ref, bmu_ref, bd1_ref, bd2_ref, bo_ref,
                  W1_hbm, W2_hbm, Wmu_hbm, cb_hbm, Wd1_hbm, Wd2_hbm, Wo_hbm,
                  out_ref,
                  w1_s, w2_s, wmu_s, cb_s, wd1_s, wd2_s, wo_s):
    def run(sem_enc, sem_cb, sem_dec):
        enc_copies = [pltpu.make_async_copy(W1_hbm, w1_s, sem_enc.at[0]),
                      pltpu.make_async_copy(W2_hbm, w2_s, sem_enc.at[1]),
                      pltpu.make_async_copy(Wmu_hbm, wmu_s, sem_enc.at[2])]

        def cb_copy(c):
            sl = pl.ds(c * CB_ROWS, CB_ROWS)
            return pltpu.make_async_copy(cb_hbm.at[sl, :], cb_s.at[sl, :],
                                         sem_cb.at[c])

        dec_copies = [pltpu.make_async_copy(Wd1_hbm, wd1_s, sem_dec.at[0]),
                      pltpu.make_async_copy(Wd2_hbm, wd2_s, sem_dec.at[1]),
                      pltpu.make_async_copy(Wo_hbm, wo_s, sem_dec.at[2])]

        for cp in enc_copies:
            cp.start()
        for c in range(CB_CHUNKS):
            cb_copy(c).start()
        for cp in dec_copies:
            cp.start()

        for cp in enc_copies:
            cp.wait()
        x = x_ref[0:1, :]
        h1 = jnp.maximum(
            jnp.dot(x, w1_s[...], preferred_element_type=jnp.float32)
            + b1_ref[...], 0.0)
        h2 = jnp.maximum(
            jnp.dot(h1, w2_s[...], preferred_element_type=jnp.float32)
            + b2_ref[...], 0.0)
        enc = (jnp.dot(h2, wmu_s[...], preferred_element_type=jnp.float32)
               + bmu_ref[...])

        minv = jnp.inf
        mini = jnp.int32(0)
        for c in range(CB_CHUNKS):
            cb_copy(c).wait()
            cb = cb_s[c * CB_ROWS:(c + 1) * CB_ROWS, :]
            cb2 = jnp.sum(cb * cb, axis=1, keepdims=True)
            scores = lax.dot_general(cb, enc, (((1,), (1,)), ((), ())),
                                     preferred_element_type=jnp.float32)
            dist = cb2 - 2.0 * scores
            m = jnp.min(dist)
            ids = lax.broadcasted_iota(jnp.int32, (CB_ROWS, 1), 0) \
                + jnp.int32(c * CB_ROWS)
            idxc = jnp.min(jnp.where(dist == m, ids, jnp.int32(K)))
            better = m < minv
            mini = jnp.where(better, idxc, mini)
            minv = jnp.where(better, m, minv)

        q = cb_s[pl.ds(mini, 1), :]
        mse_vq = jnp.mean((q - enc) ** 2)
        vq_loss = (1.0 + BETA) * mse_vq

        for cp in dec_copies:
            cp.wait()
        d1 = jnp.maximum(
            jnp.dot(q, wd1_s[...], preferred_element_type=jnp.float32)
            + bd1_ref[...], 0.0)
        d2 = jnp.maximum(
            jnp.dot(d1, wd2_s[...], preferred_element_type=jnp.float32)
            + bd2_ref[...], 0.0)
        r = jnp.tanh(
            jnp.dot(d2, wo_s[...], preferred_element_type=jnp.float32)
            + bo_ref[...])

        rows = lax.broadcasted_iota(jnp.int32, (8, ACTION_DIM), 0)
        out_ref[...] = jnp.where(rows == 0, jnp.broadcast_to(r, (8, ACTION_DIM)),
                                 vq_loss)

    pl.run_scoped(run,
                  sem_enc=pltpu.SemaphoreType.DMA((3,)),
                  sem_cb=pltpu.SemaphoreType.DMA((CB_CHUNKS,)),
                  sem_dec=pltpu.SemaphoreType.DMA((3,)))


def _combine_body(sc_ref, tc_ref, out_ref):
    colsum = jnp.sum(sc_ref[:, 0, :], axis=0, keepdims=True)
    ss = jnp.sum(sc_ref[:, 1, :])
    r = tc_ref[0:1, :]
    vq_loss = tc_ref[1, 0]
    bf = jnp.float32(B)
    recons = (bf * jnp.sum(r * r) - 2.0 * jnp.sum(r * colsum) + ss) \
        / (bf * jnp.float32(ACTION_DIM))
    total = recons + vq_loss
    lanes = lax.broadcasted_iota(jnp.int32, (8, 128), 1)
    out_ref[...] = jnp.where(
        lanes == 0, total,
        jnp.where(lanes == 1, recons,
                  jnp.where(lanes == 2, vq_loss, 0.0)))


def kernel(action, W_enc1, b_enc1, W_enc2, b_enc2, W_mu, b_mu, codebook,
           W_dec1, b_dec1, W_dec2, b_dec2, W_out, b_out):
    sc_partials = _sc_reduce(action)

    small = lambda shape: pl.BlockSpec(shape, lambda i: (0, 0))
    hbm = pl.BlockSpec(memory_space=pl.ANY)
    tc_out = pl.pallas_call(
        _tc_main_body,
        grid=(1,),
        in_specs=[
            pl.BlockSpec((8, ACTION_DIM), lambda i: (0, 0)),
            small((1, H)), small((1, H)), small((1, D)),
            small((1, H)), small((1, H)), small((1, ACTION_DIM)),
            hbm, hbm, hbm, hbm, hbm, hbm, hbm,
        ],
        out_specs=pl.BlockSpec((8, ACTION_DIM), lambda i: (0, 0)),
        out_shape=jax.ShapeDtypeStruct((8, ACTION_DIM), jnp.float32),
        scratch_shapes=[
            pltpu.VMEM((ACTION_DIM, H), jnp.float32),
            pltpu.VMEM((H, H), jnp.float32),
            pltpu.VMEM((H, D), jnp.float32),
            pltpu.VMEM((K, D), jnp.float32),
            pltpu.VMEM((D, H), jnp.float32),
            pltpu.VMEM((H, H), jnp.float32),
            pltpu.VMEM((H, ACTION_DIM), jnp.float32),
        ],
        compiler_params=pltpu.CompilerParams(
            vmem_limit_bytes=100 * 1024 * 1024,
        ),
    )(action, b_enc1.reshape(1, H), b_enc2.reshape(1, H), b_mu.reshape(1, D),
      b_dec1.reshape(1, H), b_dec2.reshape(1, H), b_out.reshape(1, ACTION_DIM),
      W_enc1, W_enc2, W_mu, codebook, W_dec1, W_dec2, W_out)

    res = pl.pallas_call(
        _combine_body,
        out_shape=jax.ShapeDtypeStruct((8, 128), jnp.float32),
    )(sc_partials, tc_out)
    return (res[0, 0], res[0, 1], res[0, 2])


# speedup vs baseline: 1.9392x; 1.9392x over previous
import jax
import jax.numpy as jnp
from jax import lax
from jax.experimental import pallas as pl
from jax.experimental.pallas import tpu as pltpu

B = 16384
ACTION_DIM = 256
H = 1024
D = 256
K = 8192
BETA = 0.25

GRID = 16
BLOCK_B = B // (2 * GRID)
CB_CHUNKS = 8
CB_ROWS = K // CB_CHUNKS


def _body(a_ref, a2_ref, b1_ref, b2_ref, bmu_ref, bd1_ref, bd2_ref, bo_ref,
          W1_hbm, W2_hbm, Wmu_hbm, cb_hbm, Wd1_hbm, Wd2_hbm, Wo_hbm,
          out_ref,
          w1_s, w2_s, wmu_s, cb_s, wd1_s, wd2_s, wo_s,
          x_s, colsum_s, sqsum_s, enc_s, q_s, d1_s, d2_s, r_s,
          minv_s, mini_s,
          sem_enc, sem_cb, sem_dec):
    i = pl.program_id(0)

    def enc_copies():
        return [pltpu.make_async_copy(W1_hbm, w1_s, sem_enc.at[0]),
                pltpu.make_async_copy(W2_hbm, w2_s, sem_enc.at[1]),
                pltpu.make_async_copy(Wmu_hbm, wmu_s, sem_enc.at[2])]

    def cb_copy(c):
        sl = pl.ds(c * CB_ROWS, CB_ROWS)
        return pltpu.make_async_copy(cb_hbm.at[sl, :], cb_s.at[sl, :],
                                     sem_cb.at[c])

    def dec_copies():
        return [pltpu.make_async_copy(Wd1_hbm, wd1_s, sem_dec.at[0]),
                pltpu.make_async_copy(Wd2_hbm, wd2_s, sem_dec.at[1]),
                pltpu.make_async_copy(Wo_hbm, wo_s, sem_dec.at[2])]

    @pl.when(i == 0)
    def _init():
        x_s[...] = a_ref[0:1, :]
        colsum_s[...] = jnp.zeros_like(colsum_s)
        sqsum_s[...] = jnp.zeros_like(sqsum_s)
        minv_s[0, 0] = jnp.inf
        mini_s[0, 0] = 0
        for cp in enc_copies():
            cp.start()
        for c in range(CB_CHUNKS):
            cb_copy(c).start()
        for cp in dec_copies():
            cp.start()

    a = a_ref[...]
    a2 = a2_ref[...]
    colsum_s[...] += (jnp.sum(a, axis=0, keepdims=True)
                      + jnp.sum(a2, axis=0, keepdims=True))
    sqsum_s[...] += (jnp.sum(a * a, axis=0, keepdims=True)
                     + jnp.sum(a2 * a2, axis=0, keepdims=True))

    @pl.when(i == 2)
    def _encode():
        for cp in enc_copies():
            cp.wait()
        x = x_s[...]
        h1 = jnp.maximum(
            jnp.dot(x, w1_s[...], preferred_element_type=jnp.float32)
            + b1_ref[...], 0.0)
        h2 = jnp.maximum(
            jnp.dot(h1, w2_s[...], preferred_element_type=jnp.float32)
            + b2_ref[...], 0.0)
        enc_s[...] = (jnp.dot(h2, wmu_s[...],
                              preferred_element_type=jnp.float32)
                      + bmu_ref[...])

    for c in range(CB_CHUNKS):
        @pl.when(i == 3 + c)
        def _chunk(c=c):
            cb_copy(c).wait()
            cb = cb_s[c * CB_ROWS:(c + 1) * CB_ROWS, :]
            enc = enc_s[...]
            cb2 = jnp.sum(cb * cb, axis=1, keepdims=True)
            scores = lax.dot_general(cb, enc, (((1,), (1,)), ((), ())),
                                     preferred_element_type=jnp.float32)
            dist = cb2 - 2.0 * scores
            m = jnp.min(dist)
            ids = lax.broadcasted_iota(jnp.int32, (CB_ROWS, 1), 0) \
                + jnp.int32(c * CB_ROWS)
            idxc = jnp.min(jnp.where(dist == m, ids, jnp.int32(K)))
            better = m < minv_s[0, 0]
            mini_s[0, 0] = jnp.where(better, idxc, mini_s[0, 0])
            minv_s[0, 0] = jnp.where(better, m, minv_s[0, 0])

    @pl.when(i == 3 + CB_CHUNKS)
    def _gather_q():
        idx = mini_s[0, 0]
        q_s[...] = cb_s[pl.ds(idx, 1), :]

    @pl.when(i == 4 + CB_CHUNKS)
    def _dec1():
        for cp in dec_copies():
            cp.wait()
        d1_s[...] = jnp.maximum(
            jnp.dot(q_s[...], wd1_s[...], preferred_element_type=jnp.float32)
            + bd1_ref[...], 0.0)

    @pl.when(i == 5 + CB_CHUNKS)
    def _dec2():
        d2_s[...] = jnp.maximum(
            jnp.dot(d1_s[...], wd2_s[...], preferred_element_type=jnp.float32)
            + bd2_ref[...], 0.0)

    @pl.when(i == 6 + CB_CHUNKS)
    def _dec3():
        r_s[...] = jnp.tanh(
            jnp.dot(d2_s[...], wo_s[...], preferred_element_type=jnp.float32)
            + bo_ref[...])

    @pl.when(i == pl.num_programs(0) - 1)
    def _finish():
        enc = enc_s[...]
        q = q_s[...]
        mse_vq = jnp.mean((q - enc) ** 2)
        vq_loss = (1.0 + BETA) * mse_vq

        r = r_s[...]
        colsum = colsum_s[...]
        ss = jnp.sum(sqsum_s[...])
        bf = jnp.float32(B)
        recons = (bf * jnp.sum(r * r) - 2.0 * jnp.sum(r * colsum) + ss) \
            / (bf * jnp.float32(ACTION_DIM))
        total = recons + vq_loss

        lanes = lax.broadcasted_iota(jnp.int32, (8, 128), 1)
        out_ref[...] = jnp.where(
            lanes == 0, total,
            jnp.where(lanes == 1, recons,
                      jnp.where(lanes == 2, vq_loss, 0.0)))


def kernel(action, W_enc1, b_enc1, W_enc2, b_enc2, W_mu, b_mu, codebook,
           W_dec1, b_dec1, W_dec2, b_dec2, W_out, b_out):
    small = lambda shape: pl.BlockSpec(shape, lambda i: (0, 0))
    hbm = pl.BlockSpec(memory_space=pl.ANY)
    res = pl.pallas_call(
        _body,
        grid=(GRID,),
        in_specs=[
            pl.BlockSpec((BLOCK_B, ACTION_DIM), lambda i: (i, 0)),
            pl.BlockSpec((BLOCK_B, ACTION_DIM), lambda i: (GRID + i, 0)),
            small((1, H)), small((1, H)), small((1, D)),
            small((1, H)), small((1, H)), small((1, ACTION_DIM)),
            hbm, hbm, hbm, hbm, hbm, hbm, hbm,
        ],
        out_specs=pl.BlockSpec((8, 128), lambda i: (0, 0)),
        out_shape=jax.ShapeDtypeStruct((8, 128), jnp.float32),
        scratch_shapes=[
            pltpu.VMEM((ACTION_DIM, H), jnp.float32),
            pltpu.VMEM((H, H), jnp.float32),
            pltpu.VMEM((H, D), jnp.float32),
            pltpu.VMEM((K, D), jnp.float32),
            pltpu.VMEM((D, H), jnp.float32),
            pltpu.VMEM((H, H), jnp.float32),
            pltpu.VMEM((H, ACTION_DIM), jnp.float32),
            pltpu.VMEM((1, ACTION_DIM), jnp.float32),
            pltpu.VMEM((1, ACTION_DIM), jnp.float32),
            pltpu.VMEM((1, ACTION_DIM), jnp.float32),
            pltpu.VMEM((1, D), jnp.float32),
            pltpu.VMEM((1, D), jnp.float32),
            pltpu.VMEM((1, H), jnp.float32),
            pltpu.VMEM((1, H), jnp.float32),
            pltpu.VMEM((1, ACTION_DIM), jnp.float32),
            pltpu.SMEM((1, 1), jnp.float32),
            pltpu.SMEM((1, 1), jnp.int32),
            pltpu.SemaphoreType.DMA((3,)),
            pltpu.SemaphoreType.DMA((CB_CHUNKS,)),
            pltpu.SemaphoreType.DMA((3,)),
        ],
        compiler_params=pltpu.CompilerParams(
            vmem_limit_bytes=100 * 1024 * 1024,
        ),
    )(action, action, b_enc1.reshape(1, H), b_enc2.reshape(1, H), b_mu.reshape(1, D),
      b_dec1.reshape(1, H), b_dec2.reshape(1, H), b_out.reshape(1, ACTION_DIM),
      W_enc1, W_enc2, W_mu, codebook, W_dec1, W_dec2, W_out)
    return (res[0, 0], res[0, 1], res[0, 2])
